# baseline (speedup 1.0000x reference)
import functools

import jax
import jax.numpy as jnp
from jax import lax
from jax.experimental import pallas as pl
from jax.experimental.pallas import tpu as pltpu
from jax.experimental.pallas import tpu_sc as plsc

N = 10000
E = 320000
D = 128
NC = 2
NS = 16
NW = NC * NS
CH = 128
NCH = 80
E_PAD = NW * NCH * CH
NCHT = NW * NCH
NCH0 = 40
NCH1 = 120
NCHMAX = 120
ACC_ROWS = 10240
RPT = ACC_ROWS // NS
HW = 128
BR = 1000

_MESH = plsc.VectorSubcoreMesh(core_axis_name="c", subcore_axis_name="s")


@functools.partial(
    pl.kernel,
    out_type=jax.ShapeDtypeStruct((NC * ACC_ROWS, HW), jnp.float32),
    mesh=_MESH,
    scratch_types=[
        pltpu.VMEM((NCH, CH), jnp.int32),
        pltpu.VMEM((CH, HW), jnp.float32),
        pltpu.VMEM_SHARED((ACC_ROWS, HW), jnp.float32),
    ],
)
def _hist_kernel(dst_hbm, ones_hbm, zeros_hbm, out_hbm, dst_v, ones_v, acc):
    cid = lax.axis_index("c")
    sid = lax.axis_index("s")
    wid = cid * NS + sid
    pltpu.sync_copy(zeros_hbm.at[pl.ds(sid * RPT, RPT)],
                    acc.at[pl.ds(sid * RPT, RPT)])
    pltpu.sync_copy(dst_hbm.at[wid], dst_v)
    pltpu.sync_copy(ones_hbm, ones_v)
    plsc.subcore_barrier()

    def body(c, carry):
        pltpu.sync_copy(ones_v, acc.at[dst_v.at[c]], add=True)
        return carry

    lax.fori_loop(0, NCH, body, 0)
    plsc.subcore_barrier()
    pltpu.sync_copy(acc.at[pl.ds(sid * RPT, RPT)],
                    out_hbm.at[pl.ds(cid * ACC_ROWS + sid * RPT, RPT)])


@functools.partial(
    pl.kernel,
    out_type=jax.ShapeDtypeStruct((NC * ACC_ROWS, D), jnp.float32),
    mesh=_MESH,
    scratch_types=[
        pltpu.VMEM((NCHMAX, CH), jnp.int32),
        pltpu.VMEM((NCHMAX, CH), jnp.int32),
        pltpu.VMEM((CH, D), jnp.float32),
        pltpu.VMEM_SHARED((ACC_ROWS, D), jnp.float32),
    ],
)
def _gather_scatter_kernel(table_hbm, src_hbm, dst_hbm, zeros_hbm, out_hbm,
                           src_v, dst_v, rows_v, acc):
    cid = lax.axis_index("c")
    sid = lax.axis_index("s")
    pltpu.sync_copy(zeros_hbm.at[pl.ds(sid * RPT, RPT)],
                    acc.at[pl.ds(sid * RPT, RPT)])
    n_my = jnp.where(cid == 0, NCH0, NCH1)
    start = jnp.where(cid == 0, sid * NCH0, NS * NCH0 + sid * NCH1)

    @pl.when(cid == 0)
    def _():
        pltpu.sync_copy(src_hbm.at[pl.ds(start, NCH0)],
                        src_v.at[pl.ds(0, NCH0)])
        pltpu.sync_copy(dst_hbm.at[pl.ds(start, NCH0)],
                        dst_v.at[pl.ds(0, NCH0)])

    @pl.when(cid == 1)
    def _():
        pltpu.sync_copy(src_hbm.at[pl.ds(start, NCH1)],
                        src_v.at[pl.ds(0, NCH1)])
        pltpu.sync_copy(dst_hbm.at[pl.ds(start, NCH1)],
                        dst_v.at[pl.ds(0, NCH1)])

    plsc.subcore_barrier()

    def body(c, carry):
        pltpu.sync_copy(table_hbm.at[src_v.at[c]], rows_v)
        pltpu.sync_copy(rows_v, acc.at[dst_v.at[c]], add=True)
        return carry

    lax.fori_loop(0, n_my, body, 0)
    plsc.subcore_barrier()
    pltpu.sync_copy(acc.at[pl.ds(sid * RPT, RPT)],
                    out_hbm.at[pl.ds(cid * ACC_ROWS + sid * RPT, RPT)])


def _dinv(hist_ref):
    deg = hist_ref[0, :, 0:1] + hist_ref[1, :, 0:1] + 1.0
    return lax.rsqrt(deg)


def _mm_scale_body(hist_ref, x_ref, w_ref, o_ref):
    h = jnp.dot(x_ref[...], w_ref[...], preferred_element_type=jnp.float32)
    o_ref[...] = h * _dinv(hist_ref)


def _mid_body(hist_ref, s_ref, hp_ref, b_ref, w_ref, o_ref):
    dinv = _dinv(hist_ref)
    z = (s_ref[0] + s_ref[1] + hp_ref[...]) * dinv + b_ref[...]
    z = jnp.maximum(z, 0.0)
    h = jnp.dot(z, w_ref[...], preferred_element_type=jnp.float32)
    o_ref[...] = h * dinv


def _fin_body(hist_ref, s_ref, hp_ref, b_ref, o_ref):
    o_ref[...] = ((s_ref[0] + s_ref[1] + hp_ref[...]) * _dinv(hist_ref)
                  + b_ref[...])


_hist_spec = pl.BlockSpec((NC, BR, HW), lambda i: (0, i, 0))
_row_spec = pl.BlockSpec((BR, D), lambda i: (i, 0))
_s_spec = pl.BlockSpec((NC, BR, D), lambda i: (0, i, 0))
_w_spec = pl.BlockSpec((D, D), lambda i: (0, 0))
_b_spec = pl.BlockSpec((1, D), lambda i: (0, 0))
_out_struct = jax.ShapeDtypeStruct((N, D), jnp.float32)

_mm_scale = pl.pallas_call(
    _mm_scale_body, grid=(N // BR,),
    in_specs=[_hist_spec, _row_spec, _w_spec],
    out_specs=_row_spec, out_shape=_out_struct)

_mid = pl.pallas_call(
    _mid_body, grid=(N // BR,),
    in_specs=[_hist_spec, _s_spec, _row_spec, _b_spec, _w_spec],
    out_specs=_row_spec, out_shape=_out_struct)

_fin = pl.pallas_call(
    _fin_body, grid=(N // BR,),
    in_specs=[_hist_spec, _s_spec, _row_spec, _b_spec],
    out_specs=_row_spec, out_shape=_out_struct)


def kernel(x, edge_index, W1, b1, W2, b2):
    ei = edge_index.astype(jnp.int32)
    pad = E_PAD - E
    src_flat = jnp.concatenate([ei[0], jnp.zeros((pad,), jnp.int32)])
    dst_flat = jnp.concatenate([ei[1], jnp.full((pad,), N, jnp.int32)])
    src_p = src_flat.reshape(NCHT, CH)
    dst_p = dst_flat.reshape(NCHT, CH)
    dst_p32 = dst_flat.reshape(NW, NCH, CH)
    ones_h = jnp.ones((CH, HW), jnp.float32)
    zeros_h = jnp.zeros((ACC_ROWS, HW), jnp.float32)
    zeros_d = jnp.zeros((ACC_ROWS, D), jnp.float32)
    b1r = b1.reshape(1, D)
    b2r = b2.reshape(1, D)

    hist = _hist_kernel(dst_p32, ones_h, zeros_h).reshape(NC, ACC_ROWS, HW)
    h1p = _mm_scale(hist, x, W1)
    s1 = _gather_scatter_kernel(h1p, src_p, dst_p, zeros_d)
    h2p = _mid(hist, s1.reshape(NC, ACC_ROWS, D), h1p, b1r, W2)
    s2 = _gather_scatter_kernel(h2p, src_p, dst_p, zeros_d)
    return _fin(hist, s2.reshape(NC, ACC_ROWS, D), h2p, b2r)


# speedup vs baseline: 1.3075x; 1.3075x over previous
import functools

import jax
import jax.numpy as jnp
from jax import lax
from jax.experimental import pallas as pl
from jax.experimental.pallas import tpu as pltpu
from jax.experimental.pallas import tpu_sc as plsc

N = 10000
E = 320000
D = 128
NC = 2
NS = 16
NW = NC * NS
CH = 128
NCH = 80
E_PAD = NW * NCH * CH
NCHT = NW * NCH
NCH0 = 120
NCH1 = 40
NCHMAX = 120
ACC_ROWS = 10240
RPT = ACC_ROWS // NS
HW = 128
BR = 1000

_MESH = plsc.VectorSubcoreMesh(core_axis_name="c", subcore_axis_name="s")


@functools.partial(
    pl.kernel,
    out_type=jax.ShapeDtypeStruct((NC * ACC_ROWS, HW), jnp.float32),
    mesh=_MESH,
    scratch_types=[
        pltpu.VMEM((NCH, CH), jnp.int32),
        pltpu.VMEM((CH, HW), jnp.float32),
        pltpu.VMEM_SHARED((ACC_ROWS, HW), jnp.float32),
    ],
)
def _hist_kernel(dst_hbm, ones_hbm, zeros_hbm, out_hbm, dst_v, ones_v, acc):
    cid = lax.axis_index("c")
    sid = lax.axis_index("s")
    wid = cid * NS + sid
    pltpu.sync_copy(zeros_hbm.at[pl.ds(sid * RPT, RPT)],
                    acc.at[pl.ds(sid * RPT, RPT)])
    pltpu.sync_copy(dst_hbm.at[wid], dst_v)
    pltpu.sync_copy(ones_hbm, ones_v)
    plsc.subcore_barrier()

    def body(c, carry):
        pltpu.sync_copy(ones_v, acc.at[dst_v.at[c]], add=True)
        return carry

    lax.fori_loop(0, NCH, body, 0)
    plsc.subcore_barrier()
    pltpu.sync_copy(acc.at[pl.ds(sid * RPT, RPT)],
                    out_hbm.at[pl.ds(cid * ACC_ROWS + sid * RPT, RPT)])


@functools.partial(
    pl.kernel,
    out_type=jax.ShapeDtypeStruct((NC * ACC_ROWS, D), jnp.float32),
    mesh=_MESH,
    scratch_types=[
        pltpu.VMEM((NCHMAX, CH), jnp.int32),
        pltpu.VMEM((NCHMAX, CH), jnp.int32),
        pltpu.VMEM((CH, D), jnp.float32),
        pltpu.VMEM_SHARED((ACC_ROWS, D), jnp.float32),
    ],
)
def _gather_scatter_kernel(table_hbm, src_hbm, dst_hbm, zeros_hbm, out_hbm,
                           src_v, dst_v, rows_v, acc):
    cid = lax.axis_index("c")
    sid = lax.axis_index("s")
    pltpu.sync_copy(zeros_hbm.at[pl.ds(sid * RPT, RPT)],
                    acc.at[pl.ds(sid * RPT, RPT)])
    n_my = jnp.where(cid == 0, NCH0, NCH1)
    start = jnp.where(cid == 0, sid * NCH0, NS * NCH0 + sid * NCH1)

    @pl.when(cid == 0)
    def _():
        pltpu.sync_copy(src_hbm.at[pl.ds(start, NCH0)],
                        src_v.at[pl.ds(0, NCH0)])
        pltpu.sync_copy(dst_hbm.at[pl.ds(start, NCH0)],
                        dst_v.at[pl.ds(0, NCH0)])

    @pl.when(cid == 1)
    def _():
        pltpu.sync_copy(src_hbm.at[pl.ds(start, NCH1)],
                        src_v.at[pl.ds(0, NCH1)])
        pltpu.sync_copy(dst_hbm.at[pl.ds(start, NCH1)],
                        dst_v.at[pl.ds(0, NCH1)])

    plsc.subcore_barrier()

    def body(c, carry):
        pltpu.sync_copy(table_hbm.at[src_v.at[c]], rows_v)
        pltpu.sync_copy(rows_v, acc.at[dst_v.at[c]], add=True)
        return carry

    lax.fori_loop(0, n_my, body, 0)
    plsc.subcore_barrier()
    pltpu.sync_copy(acc.at[pl.ds(sid * RPT, RPT)],
                    out_hbm.at[pl.ds(cid * ACC_ROWS + sid * RPT, RPT)])


def _dinv(hist_ref):
    deg = hist_ref[0, :, 0:1] + hist_ref[1, :, 0:1] + 1.0
    return lax.rsqrt(deg)


def _mm_scale_body(hist_ref, x_ref, w_ref, o_ref):
    h = jnp.dot(x_ref[...], w_ref[...], preferred_element_type=jnp.float32)
    o_ref[...] = h * _dinv(hist_ref)


def _mid_body(hist_ref, s_ref, hp_ref, b_ref, w_ref, o_ref):
    dinv = _dinv(hist_ref)
    z = (s_ref[0] + s_ref[1] + hp_ref[...]) * dinv + b_ref[...]
    z = jnp.maximum(z, 0.0)
    h = jnp.dot(z, w_ref[...], preferred_element_type=jnp.float32)
    o_ref[...] = h * dinv


def _fin_body(hist_ref, s_ref, hp_ref, b_ref, o_ref):
    o_ref[...] = ((s_ref[0] + s_ref[1] + hp_ref[...]) * _dinv(hist_ref)
                  + b_ref[...])


_hist_spec = pl.BlockSpec((NC, BR, HW), lambda i: (0, i, 0))
_row_spec = pl.BlockSpec((BR, D), lambda i: (i, 0))
_s_spec = pl.BlockSpec((NC, BR, D), lambda i: (0, i, 0))
_w_spec = pl.BlockSpec((D, D), lambda i: (0, 0))
_b_spec = pl.BlockSpec((1, D), lambda i: (0, 0))
_out_struct = jax.ShapeDtypeStruct((N, D), jnp.float32)

_mm_scale = pl.pallas_call(
    _mm_scale_body, grid=(N // BR,),
    in_specs=[_hist_spec, _row_spec, _w_spec],
    out_specs=_row_spec, out_shape=_out_struct)

_mid = pl.pallas_call(
    _mid_body, grid=(N // BR,),
    in_specs=[_hist_spec, _s_spec, _row_spec, _b_spec, _w_spec],
    out_specs=_row_spec, out_shape=_out_struct)

_fin = pl.pallas_call(
    _fin_body, grid=(N // BR,),
    in_specs=[_hist_spec, _s_spec, _row_spec, _b_spec],
    out_specs=_row_spec, out_shape=_out_struct)


def kernel(x, edge_index, W1, b1, W2, b2):
    ei = edge_index.astype(jnp.int32)
    pad = E_PAD - E
    src_flat = jnp.concatenate([ei[0], jnp.zeros((pad,), jnp.int32)])
    dst_flat = jnp.concatenate([ei[1], jnp.full((pad,), N, jnp.int32)])
    src_p = src_flat.reshape(NCHT, CH)
    dst_p = dst_flat.reshape(NCHT, CH)
    dst_p32 = dst_flat.reshape(NW, NCH, CH)
    ones_h = jnp.ones((CH, HW), jnp.float32)
    zeros_h = jnp.zeros((ACC_ROWS, HW), jnp.float32)
    zeros_d = jnp.zeros((ACC_ROWS, D), jnp.float32)
    b1r = b1.reshape(1, D)
    b2r = b2.reshape(1, D)

    hist = _hist_kernel(dst_p32, ones_h, zeros_h).reshape(NC, ACC_ROWS, HW)
    h1p = _mm_scale(hist, x, W1)
    s1 = _gather_scatter_kernel(h1p, src_p, dst_p, zeros_d)
    h2p = _mid(hist, s1.reshape(NC, ACC_ROWS, D), h1p, b1r, W2)
    s2 = _gather_scatter_kernel(h2p, src_p, dst_p, zeros_d)
    return _fin(hist, s2.reshape(NC, ACC_ROWS, D), h2p, b2r)


# speedup vs baseline: 1.4699x; 1.1243x over previous
import functools

import jax
import jax.numpy as jnp
from jax import lax
from jax.experimental import pallas as pl
from jax.experimental.pallas import tpu as pltpu
from jax.experimental.pallas import tpu_sc as plsc

N = 10000
E = 320000
D = 128
NC = 2
NS = 16
NW = NC * NS
CH = 128
NCH = 80
E_PAD = NW * NCH * CH
NCHT = NW * NCH
NCH0 = 136
NCH1 = 24
NCHMAX = 72
ACC_ROWS = 10240
RPT = ACC_ROWS // NS
HW = 128
BR = 1000

_MESH = plsc.VectorSubcoreMesh(core_axis_name="c", subcore_axis_name="s")


@functools.partial(
    pl.kernel,
    out_type=jax.ShapeDtypeStruct((NC * ACC_ROWS, HW), jnp.float32),
    mesh=_MESH,
    scratch_types=[
        pltpu.VMEM((NCH, CH), jnp.int32),
        pltpu.VMEM((CH, HW), jnp.float32),
        pltpu.VMEM_SHARED((ACC_ROWS, HW), jnp.float32),
    ],
)
def _hist_kernel(dst_hbm, ones_hbm, zeros_hbm, out_hbm, dst_v, ones_v, acc):
    cid = lax.axis_index("c")
    sid = lax.axis_index("s")
    wid = cid * NS + sid
    pltpu.sync_copy(zeros_hbm.at[pl.ds(sid * RPT, RPT)],
                    acc.at[pl.ds(sid * RPT, RPT)])
    pltpu.sync_copy(dst_hbm.at[wid], dst_v)
    pltpu.sync_copy(ones_hbm, ones_v)
    plsc.subcore_barrier()

    def body(c, carry):
        pltpu.sync_copy(ones_v, acc.at[dst_v.at[c]], add=True)
        return carry

    lax.fori_loop(0, NCH, body, 0)
    plsc.subcore_barrier()
    pltpu.sync_copy(acc.at[pl.ds(sid * RPT, RPT)],
                    out_hbm.at[pl.ds(cid * ACC_ROWS + sid * RPT, RPT)])


@functools.partial(
    pl.kernel,
    out_type=jax.ShapeDtypeStruct((NC * ACC_ROWS, D), jnp.float32),
    mesh=_MESH,
    scratch_types=[
        pltpu.VMEM((NCHMAX, CH), jnp.int32),
        pltpu.VMEM((NCHMAX, CH), jnp.int32),
        pltpu.VMEM((CH, D), jnp.float32),
        pltpu.VMEM_SHARED((ACC_ROWS, D), jnp.float32),
    ],
)
def _gather_scatter_kernel(table_hbm, src_hbm, dst_hbm, zeros_hbm, out_hbm,
                           src_v, dst_v, rows_v, acc):
    cid = lax.axis_index("c")
    sid = lax.axis_index("s")
    pltpu.sync_copy(zeros_hbm.at[pl.ds(sid * RPT, RPT)],
                    acc.at[pl.ds(sid * RPT, RPT)])
    def run_round(start, cnt):
        pltpu.sync_copy(src_hbm.at[pl.ds(start, cnt)],
                        src_v.at[pl.ds(0, cnt)])
        pltpu.sync_copy(dst_hbm.at[pl.ds(start, cnt)],
                        dst_v.at[pl.ds(0, cnt)])

        def body(c, carry):
            pltpu.sync_copy(table_hbm.at[src_v.at[c]], rows_v)
            pltpu.sync_copy(rows_v, acc.at[dst_v.at[c]], add=True)
            return carry

        lax.fori_loop(0, cnt, body, 0)

    @pl.when(cid == 0)
    def _():
        run_round(sid * NCH0, NCHMAX)
        run_round(sid * NCH0 + NCHMAX, NCH0 - NCHMAX)

    @pl.when(cid == 1)
    def _():
        run_round(NS * NCH0 + sid * NCH1, NCH1)

    plsc.subcore_barrier()
    pltpu.sync_copy(acc.at[pl.ds(sid * RPT, RPT)],
                    out_hbm.at[pl.ds(cid * ACC_ROWS + sid * RPT, RPT)])


def _dinv(hist_ref):
    deg = hist_ref[0, :, 0:1] + hist_ref[1, :, 0:1] + 1.0
    return lax.rsqrt(deg)


def _mm_scale_body(hist_ref, x_ref, w_ref, o_ref):
    h = jnp.dot(x_ref[...], w_ref[...], preferred_element_type=jnp.float32)
    o_ref[...] = h * _dinv(hist_ref)


def _mid_body(hist_ref, s_ref, hp_ref, b_ref, w_ref, o_ref):
    dinv = _dinv(hist_ref)
    z = (s_ref[0] + s_ref[1] + hp_ref[...]) * dinv + b_ref[...]
    z = jnp.maximum(z, 0.0)
    h = jnp.dot(z, w_ref[...], preferred_element_type=jnp.float32)
    o_ref[...] = h * dinv


def _fin_body(hist_ref, s_ref, hp_ref, b_ref, o_ref):
    o_ref[...] = ((s_ref[0] + s_ref[1] + hp_ref[...]) * _dinv(hist_ref)
                  + b_ref[...])


_hist_spec = pl.BlockSpec((NC, BR, HW), lambda i: (0, i, 0))
_row_spec = pl.BlockSpec((BR, D), lambda i: (i, 0))
_s_spec = pl.BlockSpec((NC, BR, D), lambda i: (0, i, 0))
_w_spec = pl.BlockSpec((D, D), lambda i: (0, 0))
_b_spec = pl.BlockSpec((1, D), lambda i: (0, 0))
_out_struct = jax.ShapeDtypeStruct((N, D), jnp.float32)

_mm_scale = pl.pallas_call(
    _mm_scale_body, grid=(N // BR,),
    in_specs=[_hist_spec, _row_spec, _w_spec],
    out_specs=_row_spec, out_shape=_out_struct)

_mid = pl.pallas_call(
    _mid_body, grid=(N // BR,),
    in_specs=[_hist_spec, _s_spec, _row_spec, _b_spec, _w_spec],
    out_specs=_row_spec, out_shape=_out_struct)

_fin = pl.pallas_call(
    _fin_body, grid=(N // BR,),
    in_specs=[_hist_spec, _s_spec, _row_spec, _b_spec],
    out_specs=_row_spec, out_shape=_out_struct)


def kernel(x, edge_index, W1, b1, W2, b2):
    ei = edge_index.astype(jnp.int32)
    pad = E_PAD - E
    src_flat = jnp.concatenate([ei[0], jnp.zeros((pad,), jnp.int32)])
    dst_flat = jnp.concatenate([ei[1], jnp.full((pad,), N, jnp.int32)])
    src_p = src_flat.reshape(NCHT, CH)
    dst_p = dst_flat.reshape(NCHT, CH)
    dst_p32 = dst_flat.reshape(NW, NCH, CH)
    ones_h = jnp.ones((CH, HW), jnp.float32)
    zeros_h = jnp.zeros((ACC_ROWS, HW), jnp.float32)
    zeros_d = jnp.zeros((ACC_ROWS, D), jnp.float32)
    b1r = b1.reshape(1, D)
    b2r = b2.reshape(1, D)

    hist = _hist_kernel(dst_p32, ones_h, zeros_h).reshape(NC, ACC_ROWS, HW)
    h1p = _mm_scale(hist, x, W1)
    s1 = _gather_scatter_kernel(h1p, src_p, dst_p, zeros_d)
    h2p = _mid(hist, s1.reshape(NC, ACC_ROWS, D), h1p, b1r, W2)
    s2 = _gather_scatter_kernel(h2p, src_p, dst_p, zeros_d)
    return _fin(hist, s2.reshape(NC, ACC_ROWS, D), h2p, b2r)
